# baseline (speedup 1.0000x reference)
import jax
import jax.numpy as jnp
from jax import lax
from jax.experimental import pallas as pl
from jax.experimental.pallas import tpu as pltpu
from jax.experimental.pallas import tpu_sc as plsc

TOPK = 10
_NEG = -3.0e38
_BIG = 2**31 - 1
_TN = 2048
_NTILES = 16
_CHUNK = 512
_NV = _CHUNK // 16


def _bdot(x, y, dims):
    return jax.lax.dot_general(
        x, y, (dims, ((), ())),
        preferred_element_type=jnp.float32,
        precision=jax.lax.Precision.DEFAULT)


def _scores_body(h_ref, w1_ref, b1_ref, wr_ref, s_ref):
    h0 = jnp.maximum(_bdot(h_ref[...], w1_ref[...], ((1,), (1,))) + b1_ref[...],
                     0.0)
    s_ref[0] = _bdot(wr_ref[...], h0, ((1,), (1,)))


def _sc_topk_body(scores_hbm, vals_hbm, idx_hbm, s_vmem, vbuf, ibuf):
    cid = lax.axis_index("c")
    sid = lax.axis_index("s")

    @pl.when(cid == 0)
    def _work():
        base = sid * _CHUNK
        pltpu.sync_copy(scores_hbm.at[pl.ds(base, _CHUNK)], s_vmem)
        lane = lax.iota(jnp.int32, 16)
        vs = [s_vmem[pl.ds(j * 16, 16)] for j in range(_NV)]
        iv = [lane + (base + j * 16) for j in range(_NV)]
        val_full = jnp.full((16,), _NEG, jnp.float32)
        idx_full = jnp.zeros((16,), jnp.int32)
        for t in range(TOPK):
            mx = vs[0]
            for j in range(1, _NV):
                mx = jnp.maximum(mx, vs[j])
            m = jnp.max(mx)
            cmin = jnp.where(vs[0] == m, iv[0], _BIG)
            for j in range(1, _NV):
                cmin = jnp.minimum(cmin, jnp.where(vs[j] == m, iv[j], _BIG))
            imin = jnp.min(cmin)
            val_full = jnp.where(lane == t, m, val_full)
            idx_full = jnp.where(lane == t, imin, idx_full)
            for j in range(_NV):
                vs[j] = jnp.where(iv[j] == imin, _NEG, vs[j])
        vbuf[...] = val_full
        ibuf[...] = idx_full
        pltpu.sync_copy(vbuf, vals_hbm.at[sid])
        pltpu.sync_copy(ibuf, idx_hbm.at[sid])


def _finish_body(cv_ref, ci_ref, h_any, w1_ref, b1_ref, wa_ref, ba_ref,
                 wb_ref, bb_ref, wc_ref, bc_ref, wr_ref, br_ref,
                 lr_ref, rk_ref, ta_ref,
                 hrows, sems):
    cv = cv_ref[...]
    ci = ci_ref[...]
    idxs = []
    for j in range(TOPK):
        m = jnp.max(cv)
        idx = jnp.min(jnp.where(cv == m, ci, _BIG))
        idxs.append(idx)
        cv = jnp.where((cv == m) & (ci == idx), _NEG, cv)

    for j in range(TOPK):
        pltpu.make_async_copy(
            h_any.at[pl.ds(idxs[j], 1), :], hrows.at[pl.ds(j, 1), :],
            sems.at[j]).start()
    for j in range(TOPK):
        pltpu.make_async_copy(
            h_any.at[pl.ds(idxs[j], 1), :], hrows.at[pl.ds(j, 1), :],
            sems.at[j]).wait()

    ht = hrows[0:TOPK, :]
    h0t = jnp.maximum(_bdot(ht, w1_ref[...], ((1,), (1,))) + b1_ref[...], 0.0)

    a = jnp.tanh(_bdot(h0t, wa_ref[...], ((1,), (1,))) + ba_ref[...])
    g = jax.nn.sigmoid(_bdot(h0t, wb_ref[...], ((1,), (1,))) + bb_ref[...])
    att = _bdot(wc_ref[...], a * g, ((1,), (1,))) + bc_ref[0, 0]

    e = jnp.exp(att - jnp.max(att))
    w = e / jnp.sum(e)
    ta_ref[...] = w

    m_vec = _bdot(w, h0t, ((1,), (0,)))
    m16 = m_vec.astype(jnp.bfloat16).astype(jnp.float32)
    w16 = wr_ref[...].astype(jnp.bfloat16).astype(jnp.float32)
    lr_val = jnp.sum(m16 * w16) + br_ref[0, 0]
    lr_ref[...] = jnp.full((1, 1), lr_val, jnp.float32)
    rk_ref[...] = jnp.full((1, 1), jnp.exp(lr_val), jnp.float32)


@jax.jit
def kernel(h, W1, b1, Wa, ba, Wb, bb, Wc, bc, Wr, br):
    N, E = h.shape
    H = W1.shape[0]
    grid = N // _TN

    b1r = b1.reshape(1, H)
    wrr = Wr.reshape(1, H)
    brr = br.reshape(1, 1)
    bar = ba.reshape(1, -1)
    bbr = bb.reshape(1, -1)
    bcr = bc.reshape(1, 1)

    scores = pl.pallas_call(
        _scores_body,
        grid=(grid,),
        in_specs=[
            pl.BlockSpec((_TN, E), lambda i: (i, 0)),
            pl.BlockSpec((H, E), lambda i: (0, 0)),
            pl.BlockSpec((1, H), lambda i: (0, 0)),
            pl.BlockSpec((1, H), lambda i: (0, 0)),
        ],
        out_specs=pl.BlockSpec((1, 1, _TN), lambda i: (i, 0, 0)),
        out_shape=jax.ShapeDtypeStruct((grid, 1, _TN), jnp.float32),
    )(h, W1, b1r, wrr)

    s_flat = scores.reshape(N)

    sc_topk = pl.kernel(
        _sc_topk_body,
        out_type=(
            jax.ShapeDtypeStruct((_NTILES, 16), jnp.float32),
            jax.ShapeDtypeStruct((_NTILES, 16), jnp.int32),
        ),
        mesh=plsc.VectorSubcoreMesh(core_axis_name="c", subcore_axis_name="s"),
        scratch_types=[
            pltpu.VMEM((_CHUNK,), jnp.float32),
            pltpu.VMEM((16,), jnp.float32),
            pltpu.VMEM((16,), jnp.int32),
        ],
        compiler_params=pltpu.CompilerParams(needs_layout_passes=False),
    )
    cand_vals, cand_idx = sc_topk(s_flat)

    lr, rk, ta = pl.pallas_call(
        _finish_body,
        in_specs=[
            pl.BlockSpec(memory_space=pltpu.MemorySpace.VMEM),
            pl.BlockSpec(memory_space=pltpu.MemorySpace.VMEM),
            pl.BlockSpec(memory_space=pltpu.MemorySpace.HBM),
            pl.BlockSpec(memory_space=pltpu.MemorySpace.VMEM),
            pl.BlockSpec(memory_space=pltpu.MemorySpace.VMEM),
            pl.BlockSpec(memory_space=pltpu.MemorySpace.VMEM),
            pl.BlockSpec(memory_space=pltpu.MemorySpace.VMEM),
            pl.BlockSpec(memory_space=pltpu.MemorySpace.VMEM),
            pl.BlockSpec(memory_space=pltpu.MemorySpace.VMEM),
            pl.BlockSpec(memory_space=pltpu.MemorySpace.VMEM),
            pl.BlockSpec(memory_space=pltpu.MemorySpace.VMEM),
            pl.BlockSpec(memory_space=pltpu.MemorySpace.VMEM),
            pl.BlockSpec(memory_space=pltpu.MemorySpace.VMEM),
        ],
        out_shape=(
            jax.ShapeDtypeStruct((1, 1), jnp.float32),
            jax.ShapeDtypeStruct((1, 1), jnp.float32),
            jax.ShapeDtypeStruct((1, TOPK), jnp.float32),
        ),
        scratch_shapes=[
            pltpu.VMEM((16, E), jnp.float32),
            pltpu.SemaphoreType.DMA((TOPK,)),
        ],
    )(cand_vals, cand_idx, h, W1, b1r, Wa, bar, Wb, bbr, Wc, bcr, wrr, brr)

    return lr, rk, ta


# speedup vs baseline: 1.8691x; 1.8691x over previous
import jax
import jax.numpy as jnp
from jax.experimental import pallas as pl
from jax.experimental.pallas import tpu as pltpu

TOPK = 10
_NEG = -3.0e38
_TN = 1024


def _bdot(x, y, dims):
    return jax.lax.dot_general(
        x, y, (dims, ((), ())),
        preferred_element_type=jnp.float32,
        precision=jax.lax.Precision.DEFAULT)


def _fused_body(ha_ref, hb_ref, w1a_ref, w1b_ref, b1_ref, wr_ref,
                wa_ref, ba_ref, wb_ref, bb_ref, wc_ref, bc_ref, br_ref,
                lr_ref, rk_ref, ta_ref,
                s_scr, h0_scr, rows_scr):
    i = pl.program_id(0)
    ngrid = pl.num_programs(0)

    acc = _bdot(ha_ref[...], w1a_ref[...], ((1,), (1,)))
    acc = acc + _bdot(hb_ref[...], w1b_ref[...], ((1,), (1,)))
    h0 = jnp.maximum(acc + b1_ref[...], 0.0)
    h0_scr[pl.ds(i * _TN, _TN), :] = h0
    s_scr[pl.ds(i, 1), :] = _bdot(wr_ref[...], h0, ((1,), (1,)))

    @pl.when(i == ngrid - 1)
    def _finish():
        s = s_scr[...]
        rows, cols = s.shape
        flat_id = (jax.lax.broadcasted_iota(jnp.int32, (rows, cols), 0) * cols
                   + jax.lax.broadcasted_iota(jnp.int32, (rows, cols), 1))

        sm = s
        for j in range(TOPK):
            m = jnp.max(sm)
            idx = jnp.min(jnp.where(sm == m, flat_id, jnp.int32(2**31 - 1)))
            rows_scr[pl.ds(j, 1), :] = h0_scr[pl.ds(idx, 1), :]
            sm = jnp.where(flat_id == idx, _NEG, sm)

        h0t = rows_scr[0:TOPK, :]

        a = jnp.tanh(_bdot(h0t, wa_ref[...], ((1,), (1,))) + ba_ref[...])
        g = jax.nn.sigmoid(_bdot(h0t, wb_ref[...], ((1,), (1,))) + bb_ref[...])
        att = _bdot(wc_ref[...], a * g, ((1,), (1,))) + bc_ref[0, 0]

        e = jnp.exp(att - jnp.max(att))
        w = e / jnp.sum(e)
        ta_ref[...] = w

        m_vec = _bdot(w, h0t, ((1,), (0,)))
        m16 = m_vec.astype(jnp.bfloat16).astype(jnp.float32)
        w16 = wr_ref[...].astype(jnp.bfloat16).astype(jnp.float32)
        lr_val = jnp.sum(m16 * w16) + br_ref[0, 0]
        lr_ref[...] = jnp.full((1, 1), lr_val, jnp.float32)
        rk_ref[...] = jnp.full((1, 1), jnp.exp(lr_val), jnp.float32)


@jax.jit
def kernel(h, W1, b1, Wa, ba, Wb, bb, Wc, bc, Wr, br):
    N, E = h.shape
    H = W1.shape[0]
    E2 = E // 2
    grid = N // _TN

    b1r = b1.reshape(1, H)
    wrr = Wr.reshape(1, H)
    brr = br.reshape(1, 1)
    bar = ba.reshape(1, -1)
    bbr = bb.reshape(1, -1)
    bcr = bc.reshape(1, 1)

    lr, rk, ta = pl.pallas_call(
        _fused_body,
        grid=(grid,),
        in_specs=[
            pl.BlockSpec((_TN, E2), lambda i: (i, 0)),
            pl.BlockSpec((_TN, E2), lambda i: (i, 1)),
            pl.BlockSpec((H, E2), lambda i: (0, 0)),
            pl.BlockSpec((H, E2), lambda i: (0, 1)),
            pl.BlockSpec((1, H), lambda i: (0, 0)),
            pl.BlockSpec((1, H), lambda i: (0, 0)),
            pl.BlockSpec((Wa.shape[0], H), lambda i: (0, 0)),
            pl.BlockSpec((1, Wa.shape[0]), lambda i: (0, 0)),
            pl.BlockSpec((Wb.shape[0], H), lambda i: (0, 0)),
            pl.BlockSpec((1, Wb.shape[0]), lambda i: (0, 0)),
            pl.BlockSpec((1, Wa.shape[0]), lambda i: (0, 0)),
            pl.BlockSpec((1, 1), lambda i: (0, 0)),
            pl.BlockSpec((1, 1), lambda i: (0, 0)),
        ],
        out_specs=(
            pl.BlockSpec((1, 1), lambda i: (0, 0)),
            pl.BlockSpec((1, 1), lambda i: (0, 0)),
            pl.BlockSpec((1, TOPK), lambda i: (0, 0)),
        ),
        out_shape=(
            jax.ShapeDtypeStruct((1, 1), jnp.float32),
            jax.ShapeDtypeStruct((1, 1), jnp.float32),
            jax.ShapeDtypeStruct((1, TOPK), jnp.float32),
        ),
        scratch_shapes=[
            pltpu.VMEM((grid, _TN), jnp.float32),
            pltpu.VMEM((N, H), jnp.float32),
            pltpu.VMEM((16, H), jnp.float32),
        ],
    )(h, h, W1, W1, b1r, wrr, Wa, bar, Wb, bbr, Wc, bcr, brr)

    return lr, rk, ta


# speedup vs baseline: 1.9203x; 1.0274x over previous
import jax
import jax.numpy as jnp
from jax.experimental import pallas as pl
from jax.experimental.pallas import tpu as pltpu

TOPK = 10
_NEG = -3.0e38
_TN = 2048


def _bdot(x, y, dims):
    return jax.lax.dot_general(
        x, y, (dims, ((), ())),
        preferred_element_type=jnp.float32,
        precision=jax.lax.Precision.DEFAULT)


def _fused_body(ha_ref, hb_ref, w1a_ref, w1b_ref, b1_ref, wr_ref,
                wa_ref, ba_ref, wb_ref, bb_ref, wc_ref, bc_ref, br_ref,
                lr_ref, rk_ref, ta_ref,
                s_scr, h0_scr, rows_scr):
    i = pl.program_id(0)
    ngrid = pl.num_programs(0)

    acc = _bdot(ha_ref[...], w1a_ref[...], ((1,), (1,)))
    acc = acc + _bdot(hb_ref[...], w1b_ref[...], ((1,), (1,)))
    h0 = jnp.maximum(acc + b1_ref[...], 0.0)
    h0_scr[pl.ds(i * _TN, _TN), :] = h0
    s_scr[pl.ds(i, 1), :] = _bdot(wr_ref[...], h0, ((1,), (1,)))

    @pl.when(i == ngrid - 1)
    def _finish():
        s = s_scr[...]
        rows, cols = s.shape
        flat_id = (jax.lax.broadcasted_iota(jnp.int32, (rows, cols), 0) * cols
                   + jax.lax.broadcasted_iota(jnp.int32, (rows, cols), 1))

        sm = s
        for j in range(TOPK):
            m = jnp.max(sm)
            idx = jnp.min(jnp.where(sm == m, flat_id, jnp.int32(2**31 - 1)))
            rows_scr[pl.ds(j, 1), :] = h0_scr[pl.ds(idx, 1), :]
            sm = jnp.where(flat_id == idx, _NEG, sm)

        h0t = rows_scr[0:TOPK, :]

        a = jnp.tanh(_bdot(h0t, wa_ref[...], ((1,), (1,))) + ba_ref[...])
        g = jax.nn.sigmoid(_bdot(h0t, wb_ref[...], ((1,), (1,))) + bb_ref[...])
        att = _bdot(wc_ref[...], a * g, ((1,), (1,))) + bc_ref[0, 0]

        e = jnp.exp(att - jnp.max(att))
        w = e / jnp.sum(e)
        ta_ref[...] = w

        m_vec = _bdot(w, h0t, ((1,), (0,)))
        m16 = m_vec.astype(jnp.bfloat16).astype(jnp.float32)
        w16 = wr_ref[...].astype(jnp.bfloat16).astype(jnp.float32)
        lr_val = jnp.sum(m16 * w16) + br_ref[0, 0]
        lr_ref[...] = jnp.full((1, 1), lr_val, jnp.float32)
        rk_ref[...] = jnp.full((1, 1), jnp.exp(lr_val), jnp.float32)


@jax.jit
def kernel(h, W1, b1, Wa, ba, Wb, bb, Wc, bc, Wr, br):
    N, E = h.shape
    H = W1.shape[0]
    E2 = E // 2
    grid = N // _TN

    b1r = b1.reshape(1, H)
    wrr = Wr.reshape(1, H)
    brr = br.reshape(1, 1)
    bar = ba.reshape(1, -1)
    bbr = bb.reshape(1, -1)
    bcr = bc.reshape(1, 1)

    lr, rk, ta = pl.pallas_call(
        _fused_body,
        grid=(grid,),
        in_specs=[
            pl.BlockSpec((_TN, E2), lambda i: (i, 0)),
            pl.BlockSpec((_TN, E2), lambda i: (i, 1)),
            pl.BlockSpec((H, E2), lambda i: (0, 0)),
            pl.BlockSpec((H, E2), lambda i: (0, 1)),
            pl.BlockSpec((1, H), lambda i: (0, 0)),
            pl.BlockSpec((1, H), lambda i: (0, 0)),
            pl.BlockSpec((Wa.shape[0], H), lambda i: (0, 0)),
            pl.BlockSpec((1, Wa.shape[0]), lambda i: (0, 0)),
            pl.BlockSpec((Wb.shape[0], H), lambda i: (0, 0)),
            pl.BlockSpec((1, Wb.shape[0]), lambda i: (0, 0)),
            pl.BlockSpec((1, Wa.shape[0]), lambda i: (0, 0)),
            pl.BlockSpec((1, 1), lambda i: (0, 0)),
            pl.BlockSpec((1, 1), lambda i: (0, 0)),
        ],
        out_specs=(
            pl.BlockSpec((1, 1), lambda i: (0, 0)),
            pl.BlockSpec((1, 1), lambda i: (0, 0)),
            pl.BlockSpec((1, TOPK), lambda i: (0, 0)),
        ),
        out_shape=(
            jax.ShapeDtypeStruct((1, 1), jnp.float32),
            jax.ShapeDtypeStruct((1, 1), jnp.float32),
            jax.ShapeDtypeStruct((1, TOPK), jnp.float32),
        ),
        scratch_shapes=[
            pltpu.VMEM((grid, _TN), jnp.float32),
            pltpu.VMEM((N, H), jnp.float32),
            pltpu.VMEM((16, H), jnp.float32),
        ],
    )(h, h, W1, W1, b1r, wrr, Wa, bar, Wb, bbr, Wc, bcr, brr)

    return lr, rk, ta
